# Initial kernel scaffold; baseline (speedup 1.0000x reference)
import jax
import jax.numpy as jnp
from jax.experimental import pallas as pl


def kernel(h, edge_index, conv_Wih, conv_Whh, conv_bih, conv_bhh, conv_Wself, conv_bself, conv_Wneigh, conv_bneigh, cls_W, cls_b, cls_Wout, cls_bout, cnf_W, cnf_b, cnf_Wout, cnf_bout):
    raise NotImplementedError("write your pallas kernel here")


# speedup vs baseline: 5.0716x; 5.0716x over previous
import functools

import jax
import jax.numpy as jnp
from jax import lax
from jax.experimental import pallas as pl
from jax.experimental.pallas import tpu as pltpu
from jax.experimental.pallas import tpu_sc as plsc


def _sc_gather(table, idx, chunk=400):
    n, d = table.shape
    e = idx.shape[0]
    mesh = plsc.VectorSubcoreMesh(core_axis_name="c", subcore_axis_name="s")
    nw = 32
    rows_per_w = e // nw
    assert rows_per_w * nw == e and rows_per_w % chunk == 0 and chunk % 8 == 0
    n_chunks = rows_per_w // chunk

    @functools.partial(
        pl.kernel,
        mesh=mesh,
        out_type=jax.ShapeDtypeStruct((e, d), jnp.float32),
        scratch_types=[
            pltpu.VMEM((chunk,), jnp.int32),
            pltpu.VMEM((chunk, d), jnp.float32),
            pltpu.VMEM((chunk,), jnp.int32),
            pltpu.VMEM((chunk, d), jnp.float32),
            pltpu.SemaphoreType.DMA,
            pltpu.SemaphoreType.DMA,
            pltpu.SemaphoreType.DMA,
            pltpu.SemaphoreType.DMA,
        ],
    )
    def gk(h_hbm, idx_hbm, out_hbm, idx_v0, rows_v0, idx_v1, rows_v1, gs0, gs1, ws0, ws1):
        wid = lax.axis_index("s") * 2 + lax.axis_index("c")
        base = wid * rows_per_w
        idx_v = (idx_v0, idx_v1)
        rows_v = (rows_v0, rows_v1)
        gsem = (gs0, gs1)
        wsem = (ws0, ws1)

        def start_gather(k, b):
            off = base + k * chunk
            pltpu.sync_copy(idx_hbm.at[pl.ds(off, chunk)], idx_v[b])
            return pltpu.async_copy(h_hbm.at[idx_v[b]], rows_v[b], gsem[b])

        def start_write(k, b):
            off = base + k * chunk
            return pltpu.async_copy(rows_v[b], out_hbm.at[pl.ds(off, chunk)], wsem[b])

        g = start_gather(0, 0)
        writes = [None, None]
        for k in range(n_chunks):
            b = k % 2
            nb = 1 - b
            if k + 1 < n_chunks:
                if writes[nb] is not None:
                    writes[nb].wait()
                gn = start_gather(k + 1, nb)
            g.wait()
            writes[b] = start_write(k, b)
            if k + 1 < n_chunks:
                g = gn
        for w in writes:
            if w is not None:
                w.wait()

    return gk(table, idx)


def _lstm_body(xg_ref, h_ref, wcat_ref, b_ref, wself_ref, wneigh_ref, b2_ref,
               out_ref, hst, cst):
    d = h_ref.shape[1]
    t = pl.program_id(1)
    nt = pl.num_programs(1)

    @pl.when(t == 0)
    def _init():
        hst[...] = jnp.zeros_like(hst)
        cst[...] = jnp.zeros_like(cst)

    x = xg_ref[0]
    xh = jnp.concatenate([x, hst[...]], axis=1)
    gates = jnp.dot(xh, wcat_ref[...], preferred_element_type=jnp.float32)
    gates = gates + b_ref[...]
    i = jax.nn.sigmoid(gates[:, :d])
    f = jax.nn.sigmoid(gates[:, d:2 * d])
    g = jnp.tanh(gates[:, 2 * d:3 * d])
    o = jax.nn.sigmoid(gates[:, 3 * d:])
    c = f * cst[...] + i * g
    hh = o * jnp.tanh(c)
    cst[...] = c
    hst[...] = hh

    @pl.when(t == nt - 1)
    def _fin():
        out_ref[...] = jax.nn.relu(
            jnp.dot(h_ref[...], wself_ref[...], preferred_element_type=jnp.float32)
            + jnp.dot(hh, wneigh_ref[...], preferred_element_type=jnp.float32)
            + b2_ref[...])


def _lstm_layer(h, xg, wcat, b, wselfT, wneighT, b2, bn=2000):
    n, d = h.shape
    deg = xg.shape[0]
    nb = n // bn
    return pl.pallas_call(
        _lstm_body,
        grid=(nb, deg),
        in_specs=[
            pl.BlockSpec((1, bn, d), lambda j, t: (t, j, 0)),
            pl.BlockSpec((bn, d), lambda j, t: (j, 0)),
            pl.BlockSpec((2 * d, 4 * d), lambda j, t: (0, 0)),
            pl.BlockSpec((1, 4 * d), lambda j, t: (0, 0)),
            pl.BlockSpec((d, d), lambda j, t: (0, 0)),
            pl.BlockSpec((d, d), lambda j, t: (0, 0)),
            pl.BlockSpec((1, d), lambda j, t: (0, 0)),
        ],
        out_specs=pl.BlockSpec((bn, d), lambda j, t: (j, 0)),
        out_shape=jax.ShapeDtypeStruct((n, d), jnp.float32),
        scratch_shapes=[
            pltpu.VMEM((bn, d), jnp.float32),
            pltpu.VMEM((bn, d), jnp.float32),
        ],
    )(xg, h, wcat, b, wselfT, wneighT, b2)


def _heads_body(h_ref, clsW_ref, clsb_ref, clsWo_ref, clsbo_ref,
                cnfW_ref, cnfb_ref, cnfWo_ref, cnfbo_ref, o_ref, c_ref):
    x = h_ref[...]
    oacc = x
    for i in range(5):
        oacc = jax.nn.relu(
            jnp.dot(oacc, clsW_ref[i], preferred_element_type=jnp.float32)
            + clsb_ref[i])
    o_ref[...] = jnp.dot(oacc, clsWo_ref[...],
                         preferred_element_type=jnp.float32) + clsbo_ref[...]
    cacc = x
    for i in range(5):
        cacc = jax.nn.relu(
            jnp.dot(cacc, cnfW_ref[i], preferred_element_type=jnp.float32)
            + cnfb_ref[i])
    c_ref[...] = jnp.dot(cacc, cnfWo_ref[...],
                         preferred_element_type=jnp.float32) + cnfbo_ref[...]


def _heads(h, clsWT, clsb, clsWoT, clsbo, cnfWT, cnfb, cnfWoT, cnfbo, bn=2000):
    n, d = h.shape
    nb = n // bn
    ncls = clsWoT.shape[1]
    return pl.pallas_call(
        _heads_body,
        grid=(nb,),
        in_specs=[
            pl.BlockSpec((bn, d), lambda j: (j, 0)),
            pl.BlockSpec((5, d, d), lambda j: (0, 0, 0)),
            pl.BlockSpec((5, 1, d), lambda j: (0, 0, 0)),
            pl.BlockSpec((d, ncls), lambda j: (0, 0)),
            pl.BlockSpec((1, ncls), lambda j: (0, 0)),
            pl.BlockSpec((5, d, d), lambda j: (0, 0, 0)),
            pl.BlockSpec((5, 1, d), lambda j: (0, 0, 0)),
            pl.BlockSpec((d, 1), lambda j: (0, 0)),
            pl.BlockSpec((1, 1), lambda j: (0, 0)),
        ],
        out_specs=[
            pl.BlockSpec((bn, ncls), lambda j: (j, 0)),
            pl.BlockSpec((bn, 1), lambda j: (j, 0)),
        ],
        out_shape=[
            jax.ShapeDtypeStruct((n, ncls), jnp.float32),
            jax.ShapeDtypeStruct((n, 1), jnp.float32),
        ],
    )(h, clsWT, clsb, clsWoT, clsbo, cnfWT, cnfb, cnfWoT, cnfbo)


def kernel(h, edge_index, conv_Wih, conv_Whh, conv_bih, conv_bhh, conv_Wself,
           conv_bself, conv_Wneigh, conv_bneigh, cls_W, cls_b, cls_Wout,
           cls_bout, cnf_W, cnf_b, cnf_Wout, cnf_bout):
    src = edge_index[0]
    n, d = h.shape
    deg = src.shape[0] // n
    idxT = src.reshape(n, deg).T.reshape(-1).astype(jnp.int32)

    num_layers = conv_Wih.shape[0]
    for l in range(num_layers):
        wcat = jnp.concatenate(
            [conv_Wih[l].T, conv_Whh[l].T], axis=0)
        b = (conv_bih[l] + conv_bhh[l]).reshape(1, 4 * d)
        wselfT = conv_Wself[l].T
        wneighT = conv_Wneigh[l].T
        b2 = (conv_bself[l] + conv_bneigh[l]).reshape(1, d)
        xg = _sc_gather(h, idxT).reshape(deg, n, d)
        h = _lstm_layer(h, xg, wcat, b, wselfT, wneighT, b2)

    o, c = _heads(
        h,
        jnp.transpose(cls_W, (0, 2, 1)), cls_b.reshape(5, 1, d),
        cls_Wout.T, cls_bout.reshape(1, -1),
        jnp.transpose(cnf_W, (0, 2, 1)), cnf_b.reshape(5, 1, d),
        cnf_Wout.T, cnf_bout.reshape(1, 1),
    )
    return (o, h, c)
